# baseline (speedup 1.0000x reference)
import jax
import jax.numpy as jnp
from jax import lax
from jax.experimental import pallas as pl
from jax.experimental.pallas import tpu as pltpu
from jax.experimental.pallas import tpu_sc as plsc

NU = 5000
NI = 5000
NN = 10000
HD = 128
NE = 320000
KST = 10

NC = 2
NS = 16
NW = NC * NS
CH = 64
TCH = 160
EPAD = NW * TCH * CH
NNP = 10112
RPT = NNP // NS
NB = 5
GL = 3


def _spmm_step(h, reci, recv, zeros):
    mesh = plsc.VectorSubcoreMesh(core_axis_name="c", subcore_axis_name="s",
                                  num_cores=NC, num_subcores=NS)

    def body(h_hbm, reci_hbm, recv_hbm, z_hbm, out_hbm,
             slots, vslots, bufs, didx, acc_sh, isem, gsem, ssem):
        c = lax.axis_index("c")
        s = lax.axis_index("s")
        base = (s * NC + c) * TCH

        pltpu.sync_copy(z_hbm.at[pl.ds(s * RPT, RPT)],
                        acc_sh.at[pl.ds(s * RPT, RPT)])
        plsc.subcore_barrier()

        def idx_start(j, r):
            pltpu.make_async_copy(reci_hbm.at[base + j], slots.at[r],
                                  isem).start()
            pltpu.make_async_copy(recv_hbm.at[base + j], vslots.at[r],
                                  isem).start()

        def idx_wait(r):
            pltpu.make_async_copy(reci_hbm.at[0], slots.at[r], isem).wait()
            pltpu.make_async_copy(recv_hbm.at[0], vslots.at[r], isem).wait()

        def gather_start(r):
            pltpu.make_async_copy(h_hbm.at[slots.at[r, 0]], bufs.at[r],
                                  gsem).start()

        def gather_wait():
            pltpu.make_async_copy(h_hbm.at[slots.at[0, 0]], bufs.at[0],
                                  gsem).wait()

        def consume(r):
            for g in range(CH // 16):
                dsl = pl.ds(g * 16, 16)
                didx[r, dsl] = slots[r, 1, dsl]

            def k16(k0, carry):
                vf = vslots[r, pl.ds(k0 * 16, 16)]
                for kk in range(16):
                    v = vf[kk]
                    k = k0 * 16 + kk
                    for q in range(HD // 16):
                        sl = pl.ds(q * 16, 16)
                        bufs[r, k, sl] = bufs[r, k, sl] * v
                return carry

            lax.fori_loop(0, CH // 16, k16, 0)
            pltpu.async_copy(bufs.at[r], acc_sh.at[didx.at[r]], ssem,
                             add=True)

        def scatter_wait():
            pltpu.make_async_copy(bufs.at[0], acc_sh.at[didx.at[0]],
                                  ssem).wait()

        for r in range(NB):
            idx_start(r, r)
        for j in range(GL):
            idx_wait(j)
            gather_start(j)

        def bodyN(jj, carry):
            for u in range(NB):
                j = jj * NB + u

                @pl.when(j >= NB - GL)
                def _():
                    scatter_wait()

                @pl.when(j + GL < TCH)
                def _():
                    idx_wait((u + GL) % NB)
                    gather_start((u + GL) % NB)

                gather_wait()
                consume(u)

                @pl.when(j + NB < TCH)
                def _():
                    idx_start(j + NB, u)
            return carry

        lax.fori_loop(0, TCH // NB, bodyN, 0)
        for _ in range(NB - GL):
            scatter_wait()

        plsc.subcore_barrier()
        pltpu.sync_copy(acc_sh.at[pl.ds(s * RPT, RPT)],
                        out_hbm.at[c, pl.ds(s * RPT, RPT)])

    spmm = pl.kernel(
        body,
        out_type=jax.ShapeDtypeStruct((NC, NNP, HD), jnp.float32),
        mesh=mesh,
        scratch_types=[
            pltpu.VMEM((NB, 2, CH), jnp.int32),
            pltpu.VMEM((NB, CH), jnp.float32),
            pltpu.VMEM((NB, CH, HD), jnp.float32),
            pltpu.VMEM((NB, CH), jnp.int32),
            pltpu.VMEM_SHARED((NNP, HD), jnp.float32),
            pltpu.SemaphoreType.DMA,
            pltpu.SemaphoreType.DMA,
            pltpu.SemaphoreType.DMA,
        ],
    )
    return spmm(h, reci, recv, zeros)


def _update(h, p0, p1, init, step, mscale):
    def body(st_ref, m_ref, h_ref, p0_ref, p1_ref, i_ref, o_ref):
        st = st_ref[0, 0]
        m = m_ref[0, 0]
        o_ref[...] = m * ((1.0 - st) * h_ref[...]
                          + st * (p0_ref[...] + p1_ref[...] + i_ref[...]))

    return pl.pallas_call(
        body,
        out_shape=jax.ShapeDtypeStruct((NNP, HD), jnp.float32),
    )(step, mscale, h, p0, p1, init)


def _inv_norm(xs):
    def body(x_ref, o_ref):
        x = x_ref[...]
        ss = jnp.sum(x * x, axis=1)
        o_ref[...] = jnp.full((1, 1), lax.rsqrt(jnp.max(ss)), jnp.float32)

    return pl.pallas_call(
        body,
        out_shape=jax.ShapeDtypeStruct((1, 1), jnp.float32),
    )(xs)


def kernel(edge_index, adj_vals, dt, last_xu, last_xi, user_states,
           item_states):
    src = edge_index[1].astype(jnp.int32)
    dst = edge_index[0].astype(jnp.int32)
    order = jnp.argsort(src)
    src = src[order]
    dst = dst[order]
    adj_vals = adj_vals[order]
    pad = EPAD - NE
    src2d = jnp.pad(src, (0, pad)).reshape(NW * TCH, CH)
    dst2d = jnp.pad(dst, (0, pad)).reshape(NW * TCH, CH)
    val2d = jnp.pad(adj_vals, (0, pad)).reshape(NW * TCH, CH)
    reci = jnp.stack([src2d, dst2d], axis=1)
    zeros = jnp.zeros((NNP, HD), jnp.float32)

    rpad = NNP - NN
    h = jnp.pad(jnp.concatenate([last_xu, last_xi], axis=0),
                ((0, rpad), (0, 0)))
    init = jnp.pad(jnp.concatenate([user_states, item_states], axis=0),
                   ((0, rpad), (0, 0)))
    step = (dt / KST).reshape(1, 1).astype(jnp.float32)
    one = jnp.ones((1, 1), jnp.float32)
    invn = _inv_norm(h)

    for k in range(KST):
        p = _spmm_step(h, reci, val2d, zeros)
        m = invn if k == KST - 1 else one
        h = _update(h, p[0], p[1], init, step, m)

    return h[:NU], h[NU:NN]


# speedup vs baseline: 1.2979x; 1.2979x over previous
import jax
import jax.numpy as jnp
from jax import lax
from jax.experimental import pallas as pl
from jax.experimental.pallas import tpu as pltpu
from jax.experimental.pallas import tpu_sc as plsc

NU = 5000
NI = 5000
NN = 10000
HD = 128
NE = 320000
KST = 10

NC = 2
NS = 16
NW = NC * NS
CH = 64
TCH = 160
EPAD = NW * TCH * CH
NNP = 10112
RPT = NNP // NS
NB = 5
GL = 3
SD = 4


def _spmm_step(h, reci, recv, zeros):
    mesh = plsc.VectorSubcoreMesh(core_axis_name="c", subcore_axis_name="s",
                                  num_cores=NC, num_subcores=NS)

    def body(h_hbm, reci_hbm, recv_hbm, z_hbm, out_hbm,
             slots, vslots, bufs, didx, acc_sh, isem, gsem, ssem):
        c = lax.axis_index("c")
        s = lax.axis_index("s")
        base = (s * NC + c) * TCH

        pltpu.sync_copy(z_hbm.at[pl.ds(s * RPT, RPT)],
                        acc_sh.at[pl.ds(s * RPT, RPT)])
        plsc.subcore_barrier()

        def idx_start(j, r):
            pltpu.make_async_copy(reci_hbm.at[base + j], slots.at[r],
                                  isem).start()
            pltpu.make_async_copy(recv_hbm.at[base + j], vslots.at[r],
                                  isem).start()

        def idx_wait(r):
            pltpu.make_async_copy(reci_hbm.at[0], slots.at[r], isem).wait()
            pltpu.make_async_copy(recv_hbm.at[0], vslots.at[r], isem).wait()

        def gather_start(r):
            sub = CH // SD
            for q in range(SD):
                qs = pl.ds(q * sub, sub)
                pltpu.make_async_copy(h_hbm.at[slots.at[r, 0, qs]],
                                      bufs.at[r, qs], gsem).start()

        def gather_wait():
            pltpu.make_async_copy(h_hbm.at[slots.at[0, 0]], bufs.at[0],
                                  gsem).wait()

        def consume(r):
            for g in range(CH // 16):
                dsl = pl.ds(g * 16, 16)
                didx[r, dsl] = slots[r, 1, dsl]

            def k16(k0, carry):
                vf = vslots[r, pl.ds(k0 * 16, 16)]
                for kk in range(16):
                    v = vf[kk]
                    k = k0 * 16 + kk
                    for q in range(HD // 16):
                        sl = pl.ds(q * 16, 16)
                        bufs[r, k, sl] = bufs[r, k, sl] * v
                return carry

            lax.fori_loop(0, CH // 16, k16, 0)
            pltpu.async_copy(bufs.at[r], acc_sh.at[didx.at[r]], ssem,
                             add=True)

        def scatter_wait():
            pltpu.make_async_copy(bufs.at[0], acc_sh.at[didx.at[0]],
                                  ssem).wait()

        for r in range(NB):
            idx_start(r, r)
        for j in range(GL):
            idx_wait(j)
            gather_start(j)

        def bodyN(jj, carry):
            for u in range(NB):
                j = jj * NB + u

                @pl.when(j >= NB - GL)
                def _():
                    scatter_wait()

                @pl.when(j + GL < TCH)
                def _():
                    idx_wait((u + GL) % NB)
                    gather_start((u + GL) % NB)

                gather_wait()
                consume(u)

                @pl.when(j + NB < TCH)
                def _():
                    idx_start(j + NB, u)
            return carry

        lax.fori_loop(0, TCH // NB, bodyN, 0)
        for _ in range(NB - GL):
            scatter_wait()

        plsc.subcore_barrier()
        pltpu.sync_copy(acc_sh.at[pl.ds(s * RPT, RPT)],
                        out_hbm.at[c, pl.ds(s * RPT, RPT)])

    spmm = pl.kernel(
        body,
        out_type=jax.ShapeDtypeStruct((NC, NNP, HD), jnp.float32),
        mesh=mesh,
        scratch_types=[
            pltpu.VMEM((NB, 2, CH), jnp.int32),
            pltpu.VMEM((NB, CH), jnp.float32),
            pltpu.VMEM((NB, CH, HD), jnp.float32),
            pltpu.VMEM((NB, CH), jnp.int32),
            pltpu.VMEM_SHARED((NNP, HD), jnp.float32),
            pltpu.SemaphoreType.DMA,
            pltpu.SemaphoreType.DMA,
            pltpu.SemaphoreType.DMA,
        ],
    )
    return spmm(h, reci, recv, zeros)


def _update(h, p0, p1, init, step, mscale):
    def body(st_ref, m_ref, h_ref, p0_ref, p1_ref, i_ref, o_ref):
        st = st_ref[0, 0]
        m = m_ref[0, 0]
        o_ref[...] = m * ((1.0 - st) * h_ref[...]
                          + st * (p0_ref[...] + p1_ref[...] + i_ref[...]))

    return pl.pallas_call(
        body,
        out_shape=jax.ShapeDtypeStruct((NNP, HD), jnp.float32),
    )(step, mscale, h, p0, p1, init)


def _inv_norm(xs):
    def body(x_ref, o_ref):
        x = x_ref[...]
        ss = jnp.sum(x * x, axis=1)
        o_ref[...] = jnp.full((1, 1), lax.rsqrt(jnp.max(ss)), jnp.float32)

    return pl.pallas_call(
        body,
        out_shape=jax.ShapeDtypeStruct((1, 1), jnp.float32),
    )(xs)


def kernel(edge_index, adj_vals, dt, last_xu, last_xi, user_states,
           item_states):
    src = edge_index[1].astype(jnp.int32)
    dst = edge_index[0].astype(jnp.int32)
    pad = EPAD - NE
    src2d = jnp.pad(src, (0, pad)).reshape(NW * TCH, CH)
    dst2d = jnp.pad(dst, (0, pad)).reshape(NW * TCH, CH)
    val2d = jnp.pad(adj_vals, (0, pad)).reshape(NW * TCH, CH)
    reci = jnp.stack([src2d, dst2d], axis=1)
    zeros = jnp.zeros((NNP, HD), jnp.float32)

    rpad = NNP - NN
    h = jnp.pad(jnp.concatenate([last_xu, last_xi], axis=0),
                ((0, rpad), (0, 0)))
    init = jnp.pad(jnp.concatenate([user_states, item_states], axis=0),
                   ((0, rpad), (0, 0)))
    step = (dt / KST).reshape(1, 1).astype(jnp.float32)
    one = jnp.ones((1, 1), jnp.float32)
    invn = _inv_norm(h)

    for k in range(KST):
        p = _spmm_step(h, reci, val2d, zeros)
        m = invn if k == KST - 1 else one
        h = _update(h, p[0], p[1], init, step, m)

    return h[:NU], h[NU:NN]
